# Initial kernel scaffold; baseline (speedup 1.0000x reference)
import jax
import jax.numpy as jnp
from jax.experimental import pallas as pl


def kernel(x, edge_index, batch, W1, b1, W2, b2, W3, b3, g1, be1, g2, be2, g3, be3, fW1, fb1, fW2, fb2):
    raise NotImplementedError("write your pallas kernel here")


# speedup vs baseline: 11.7524x; 11.7524x over previous
import functools

import jax
import jax.numpy as jnp
from jax import lax
from jax.experimental import pallas as pl
from jax.experimental.pallas import tpu as pltpu
from jax.experimental.pallas import tpu_sc as plsc

N = 10000
E = 320000
D = 128
G = 16

NC, NS = 2, 16
NW = NC * NS
DH = D // NC
EPT = E // NS
CH = 80
NCHUNK = EPT // CH
EPW = E // NW
NCHUNK_D = EPW // CH
NP = 10240
RPT = NP // NS
RB = 400
NBLK = N // RB

_mesh = plsc.VectorSubcoreMesh(core_axis_name="c", subcore_axis_name="s",
                               num_cores=NC, num_subcores=NS)
_sc_params = pltpu.CompilerParams(use_tc_tiling_on_sc=False)


@functools.partial(
    pl.kernel,
    out_type=(jax.ShapeDtypeStruct((NP, 16), jnp.float32),
              jax.ShapeDtypeStruct((NP, 16), jnp.float32)),
    mesh=_mesh,
    scratch_types=[
        pltpu.VMEM((NCHUNK_D, CH), jnp.int32),
        pltpu.VMEM((CH, 16), jnp.float32),
        pltpu.VMEM((RPT, 16), jnp.float32),
        pltpu.VMEM_SHARED((NP, 16), jnp.float32),
    ],
    compiler_params=_sc_params,
)
def _deg_kernel(dst_hbm, out0_hbm, out1_hbm, idx_v, ones_v, zero_v, deg_sh):
    c = lax.axis_index("c")
    s = lax.axis_index("s")
    w = c * NS + s

    def fill_zero(i, carry):
        zero_v[i] = jnp.zeros((16,), jnp.float32)
        return carry
    lax.fori_loop(0, RPT, fill_zero, 0)

    def fill_ones(i, carry):
        ones_v[i] = jnp.ones((16,), jnp.float32)
        return carry
    lax.fori_loop(0, CH, fill_ones, 0)

    pltpu.sync_copy(zero_v, deg_sh.at[pl.ds(s * RPT, RPT)])
    pltpu.sync_copy(dst_hbm.at[w], idx_v)
    plsc.subcore_barrier()

    def chunk(j, carry):
        pltpu.sync_copy(ones_v, deg_sh.at[idx_v.at[j]], add=True)
        return carry
    lax.fori_loop(0, NCHUNK_D, chunk, 0)

    plsc.subcore_barrier()

    @pl.when(c == 0)
    def _():
        pltpu.sync_copy(deg_sh.at[pl.ds(s * RPT, RPT)],
                        out0_hbm.at[pl.ds(s * RPT, RPT)])

    @pl.when(c == 1)
    def _():
        pltpu.sync_copy(deg_sh.at[pl.ds(s * RPT, RPT)],
                        out1_hbm.at[pl.ds(s * RPT, RPT)])


@functools.partial(
    pl.kernel,
    out_type=(jax.ShapeDtypeStruct((NP, DH), jnp.float32),
              jax.ShapeDtypeStruct((NP, DH), jnp.float32)),
    mesh=_mesh,
    scratch_types=[
        pltpu.VMEM((NCHUNK, CH), jnp.int32),
        pltpu.VMEM((NCHUNK, CH), jnp.int32),
        pltpu.VMEM((CH, DH), jnp.float32),
        pltpu.VMEM((128, DH), jnp.float32),
        pltpu.VMEM_SHARED((NP, DH), jnp.float32),
        pltpu.SemaphoreType.DMA,
    ],
    compiler_params=_sc_params,
)
def _mp_kernel(u0_hbm, u1_hbm, src_hbm, dst_hbm, out0_hbm, out1_hbm,
               sidx_v, didx_v, rows_v, zero_v, agg_sh, gsem):
    c = lax.axis_index("c")
    s = lax.axis_index("s")

    def fill_zero(i, carry):
        for k in range(DH // 16):
            zero_v[i, pl.ds(k * 16, 16)] = jnp.zeros((16,), jnp.float32)
        return carry
    lax.fori_loop(0, 128, fill_zero, 0)

    for k in range(RPT // 128):
        pltpu.sync_copy(zero_v, agg_sh.at[pl.ds(s * RPT + k * 128, 128)])
    pltpu.sync_copy(src_hbm.at[s], sidx_v)
    pltpu.sync_copy(dst_hbm.at[s], didx_v)
    plsc.subcore_barrier()

    def chunk(j, carry):
        @pl.when(c == 0)
        def _():
            pltpu.async_copy(u0_hbm.at[sidx_v.at[j]], rows_v, gsem).wait()

        @pl.when(c == 1)
        def _():
            pltpu.async_copy(u1_hbm.at[sidx_v.at[j]], rows_v, gsem).wait()

        pltpu.sync_copy(rows_v, agg_sh.at[didx_v.at[j]], add=True)
        return carry
    lax.fori_loop(0, NCHUNK, chunk, 0)

    plsc.subcore_barrier()

    @pl.when(c == 0)
    def _():
        pltpu.sync_copy(agg_sh.at[pl.ds(s * RPT, RPT)],
                        out0_hbm.at[pl.ds(s * RPT, RPT)])

    @pl.when(c == 1)
    def _():
        pltpu.sync_copy(agg_sh.at[pl.ds(s * RPT, RPT)],
                        out1_hbm.at[pl.ds(s * RPT, RPT)])


_HI = lax.Precision.HIGHEST


def _pre_body(deg_ref, x_ref, w_ref, u0_ref, u1_ref, dinv_ref):
    deg = deg_ref[:, 0] + deg_ref[:, 1] + 1.0
    dinv = lax.rsqrt(deg)[:, None]
    hw = jnp.dot(x_ref[...], w_ref[...], precision=_HI,
                 preferred_element_type=jnp.float32)
    u = hw * dinv
    u0_ref[...] = u[:, :DH]
    u1_ref[...] = u[:, DH:]
    dinv_ref[...] = jnp.broadcast_to(dinv, (RB, D))


def _pre_call(deg_pair, x, W):
    return pl.pallas_call(
        _pre_body,
        grid=(NBLK,),
        in_specs=[
            pl.BlockSpec((RB, NC), lambda i: (i, 0)),
            pl.BlockSpec((RB, D), lambda i: (i, 0)),
            pl.BlockSpec((D, D), lambda i: (0, 0)),
        ],
        out_specs=[
            pl.BlockSpec((RB, DH), lambda i: (i, 0)),
            pl.BlockSpec((RB, DH), lambda i: (i, 0)),
            pl.BlockSpec((RB, D), lambda i: (i, 0)),
        ],
        out_shape=[
            jax.ShapeDtypeStruct((N, DH), jnp.float32),
            jax.ShapeDtypeStruct((N, DH), jnp.float32),
            jax.ShapeDtypeStruct((N, D), jnp.float32),
        ],
    )(deg_pair, x, W)


def _post_mix(agg0_ref, agg1_ref, u0_ref, u1_ref, dinv_ref, b_ref, g_ref,
              be_ref):
    dinv = dinv_ref[...]
    msg = jnp.concatenate([agg0_ref[...] + u0_ref[...],
                           agg1_ref[...] + u1_ref[...]], axis=-1)
    t = dinv * msg + b_ref[...]
    mu = jnp.mean(t, axis=-1, keepdims=True)
    var = jnp.mean((t - mu) ** 2, axis=-1, keepdims=True)
    t = (t - mu) / jnp.sqrt(var + 1e-5) * g_ref[...] + be_ref[...]
    return jnp.where(t > 0, t, 0.01 * t)


def _mid_body(agg0_ref, agg1_ref, u0_ref, u1_ref, dinv_ref, b_ref, g_ref,
              be_ref, w_ref, un0_ref, un1_ref):
    h = _post_mix(agg0_ref, agg1_ref, u0_ref, u1_ref, dinv_ref, b_ref,
                  g_ref, be_ref)
    un = dinv_ref[...] * jnp.dot(h, w_ref[...], precision=_HI,
                                 preferred_element_type=jnp.float32)
    un0_ref[...] = un[:, :DH]
    un1_ref[...] = un[:, DH:]


def _mid_call(agg0, agg1, u0, u1, dinv, b, g, be, Wn):
    return pl.pallas_call(
        _mid_body,
        grid=(NBLK,),
        in_specs=[
            pl.BlockSpec((RB, DH), lambda i: (i, 0)),
            pl.BlockSpec((RB, DH), lambda i: (i, 0)),
            pl.BlockSpec((RB, DH), lambda i: (i, 0)),
            pl.BlockSpec((RB, DH), lambda i: (i, 0)),
            pl.BlockSpec((RB, D), lambda i: (i, 0)),
            pl.BlockSpec((1, D), lambda i: (0, 0)),
            pl.BlockSpec((1, D), lambda i: (0, 0)),
            pl.BlockSpec((1, D), lambda i: (0, 0)),
            pl.BlockSpec((D, D), lambda i: (0, 0)),
        ],
        out_specs=[
            pl.BlockSpec((RB, DH), lambda i: (i, 0)),
            pl.BlockSpec((RB, DH), lambda i: (i, 0)),
        ],
        out_shape=[
            jax.ShapeDtypeStruct((N, DH), jnp.float32),
            jax.ShapeDtypeStruct((N, DH), jnp.float32),
        ],
    )(agg0, agg1, u0, u1, dinv, b, g, be, Wn)


def _final_body(agg0_ref, agg1_ref, u0_ref, u1_ref, dinv_ref, b_ref, g_ref,
                be_ref, batch_ref, fw1_ref, fb1_ref, fw2_ref, fb2_ref,
                out_ref, sums, cnts):
    i = pl.program_id(0)
    h = _post_mix(agg0_ref, agg1_ref, u0_ref, u1_ref, dinv_ref, b_ref,
                  g_ref, be_ref)
    bt = batch_ref[0, 0, :]
    mask = (bt[None, :] == lax.broadcasted_iota(jnp.int32, (G, RB), 0))
    mask = mask.astype(jnp.float32)
    psum = jnp.dot(mask, h, precision=_HI, preferred_element_type=jnp.float32)
    pcnt = jnp.broadcast_to(jnp.sum(mask, axis=1)[:, None], (G, D))

    @pl.when(i == 0)
    def _():
        sums[...] = psum
        cnts[...] = pcnt

    @pl.when(i > 0)
    def _():
        sums[...] += psum
        cnts[...] += pcnt

    @pl.when(i == NBLK - 1)
    def _():
        pooled = sums[...] / jnp.maximum(cnts[...], 1.0)
        o = jnp.dot(pooled, fw1_ref[...], precision=_HI,
                    preferred_element_type=jnp.float32) + fb1_ref[...]
        o = jnp.dot(o, fw2_ref[...], precision=_HI,
                    preferred_element_type=jnp.float32) + fb2_ref[...]
        out_ref[...] = o


def _final_call(agg0, agg1, u0, u1, dinv, b, g, be, batch_r, fW1, fb1, fW2,
                fb2):
    return pl.pallas_call(
        _final_body,
        grid=(NBLK,),
        in_specs=[
            pl.BlockSpec((RB, DH), lambda i: (i, 0)),
            pl.BlockSpec((RB, DH), lambda i: (i, 0)),
            pl.BlockSpec((RB, DH), lambda i: (i, 0)),
            pl.BlockSpec((RB, DH), lambda i: (i, 0)),
            pl.BlockSpec((RB, D), lambda i: (i, 0)),
            pl.BlockSpec((1, D), lambda i: (0, 0)),
            pl.BlockSpec((1, D), lambda i: (0, 0)),
            pl.BlockSpec((1, D), lambda i: (0, 0)),
            pl.BlockSpec((1, 1, RB), lambda i: (i, 0, 0)),
            pl.BlockSpec((D, 256), lambda i: (0, 0)),
            pl.BlockSpec((1, 256), lambda i: (0, 0)),
            pl.BlockSpec((256, D), lambda i: (0, 0)),
            pl.BlockSpec((1, D), lambda i: (0, 0)),
        ],
        out_specs=pl.BlockSpec((G, D), lambda i: (0, 0)),
        out_shape=jax.ShapeDtypeStruct((G, D), jnp.float32),
        scratch_shapes=[
            pltpu.VMEM((G, D), jnp.float32),
            pltpu.VMEM((G, D), jnp.float32),
        ],
    )(agg0, agg1, u0, u1, dinv, b, g, be, batch_r, fW1, fb1, fW2, fb2)


def kernel(x, edge_index, batch, W1, b1, W2, b2, W3, b3, g1, be1, g2, be2,
           g3, be3, fW1, fb1, fW2, fb2):
    src = edge_index[0].reshape(NS, NCHUNK, CH)
    dst = edge_index[1].reshape(NS, NCHUNK, CH)
    dst_deg = edge_index[1].reshape(NW, NCHUNK_D, CH)
    batch_r = batch.reshape(NBLK, 1, RB)
    b1r, b2r, b3r = b1.reshape(1, D), b2.reshape(1, D), b3.reshape(1, D)
    g1r, g2r, g3r = g1.reshape(1, D), g2.reshape(1, D), g3.reshape(1, D)
    be1r, be2r, be3r = be1.reshape(1, D), be2.reshape(1, D), be3.reshape(1, D)
    fb1r, fb2r = fb1.reshape(1, 256), fb2.reshape(1, D)

    dm0, dm1 = _deg_kernel(dst_deg)
    deg_pair = jnp.stack([dm0[:N, 0], dm1[:N, 0]], axis=-1)

    u10, u11, dinv = _pre_call(deg_pair, x, W1)
    a10, a11 = _mp_kernel(u10, u11, src, dst)
    u20, u21 = _mid_call(a10, a11, u10, u11, dinv, b1r, g1r, be1r, W2)
    a20, a21 = _mp_kernel(u20, u21, src, dst)
    u30, u31 = _mid_call(a20, a21, u20, u21, dinv, b2r, g2r, be2r, W3)
    a30, a31 = _mp_kernel(u30, u31, src, dst)
    return _final_call(a30, a31, u30, u31, dinv, b3r, g3r, be3r, batch_r,
                       fW1, fb1r, fW2, fb2r)


# speedup vs baseline: 12.9495x; 1.1019x over previous
import functools

import jax
import jax.numpy as jnp
from jax import lax
from jax.experimental import pallas as pl
from jax.experimental.pallas import tpu as pltpu
from jax.experimental.pallas import tpu_sc as plsc

N = 10000
E = 320000
D = 128
G = 16

NC, NS = 2, 16
NW = NC * NS
DH = D // NC
CH = 128
NCHUNK = 158
EPAD = NS * NCHUNK * CH
CHD = 80
EPW = E // NW
NCHUNK_D = EPW // CHD
NP = 10240
RPT = NP // NS
RB = 400
NBLK = N // RB

_mesh = plsc.VectorSubcoreMesh(core_axis_name="c", subcore_axis_name="s",
                               num_cores=NC, num_subcores=NS)
_sc_params = pltpu.CompilerParams(use_tc_tiling_on_sc=False)


@functools.partial(
    pl.kernel,
    out_type=(jax.ShapeDtypeStruct((NP, 16), jnp.float32),
              jax.ShapeDtypeStruct((NP, 16), jnp.float32)),
    mesh=_mesh,
    scratch_types=[
        pltpu.VMEM((NCHUNK_D, CHD), jnp.int32),
        pltpu.VMEM((CHD, 16), jnp.float32),
        pltpu.VMEM((RPT, 16), jnp.float32),
        pltpu.VMEM_SHARED((NP, 16), jnp.float32),
    ],
    compiler_params=_sc_params,
)
def _deg_kernel(dst_hbm, out0_hbm, out1_hbm, idx_v, ones_v, zero_v, deg_sh):
    c = lax.axis_index("c")
    s = lax.axis_index("s")
    w = c * NS + s

    def fill_zero(i, carry):
        zero_v[i] = jnp.zeros((16,), jnp.float32)
        return carry
    lax.fori_loop(0, RPT, fill_zero, 0)

    def fill_ones(i, carry):
        ones_v[i] = jnp.ones((16,), jnp.float32)
        return carry
    lax.fori_loop(0, CHD, fill_ones, 0)

    pltpu.sync_copy(zero_v, deg_sh.at[pl.ds(s * RPT, RPT)])
    pltpu.sync_copy(dst_hbm.at[w], idx_v)
    plsc.subcore_barrier()

    def chunk(j, carry):
        pltpu.sync_copy(ones_v, deg_sh.at[idx_v.at[j]], add=True)
        return carry
    lax.fori_loop(0, NCHUNK_D, chunk, 0)

    plsc.subcore_barrier()

    @pl.when(c == 0)
    def _():
        pltpu.sync_copy(deg_sh.at[pl.ds(s * RPT, RPT)],
                        out0_hbm.at[pl.ds(s * RPT, RPT)])

    @pl.when(c == 1)
    def _():
        pltpu.sync_copy(deg_sh.at[pl.ds(s * RPT, RPT)],
                        out1_hbm.at[pl.ds(s * RPT, RPT)])


@functools.partial(
    pl.kernel,
    out_type=(jax.ShapeDtypeStruct((NP, DH), jnp.float32),
              jax.ShapeDtypeStruct((NP, DH), jnp.float32)),
    mesh=_mesh,
    scratch_types=[
        pltpu.VMEM((NCHUNK, CH), jnp.int32),
        pltpu.VMEM((NCHUNK, CH), jnp.int32),
        pltpu.VMEM((CH, DH), jnp.float32),
        pltpu.VMEM((CH, DH), jnp.float32),
        pltpu.VMEM((128, DH), jnp.float32),
        pltpu.VMEM_SHARED((NP, DH), jnp.float32),
        pltpu.SemaphoreType.DMA,
        pltpu.SemaphoreType.DMA,
    ],
    compiler_params=_sc_params,
)
def _mp_kernel(u0_hbm, u1_hbm, src_hbm, dst_hbm, out0_hbm, out1_hbm,
               sidx_v, didx_v, rows_a, rows_b, zero_v, agg_sh, sem_a, sem_b):
    c = lax.axis_index("c")
    s = lax.axis_index("s")

    def fill_zero(i, carry):
        for k in range(DH // 16):
            zero_v[i, pl.ds(k * 16, 16)] = jnp.zeros((16,), jnp.float32)
        return carry
    lax.fori_loop(0, 128, fill_zero, 0)

    for k in range(RPT // 128):
        pltpu.sync_copy(zero_v, agg_sh.at[pl.ds(s * RPT + k * 128, 128)])
    pltpu.sync_copy(src_hbm.at[s], sidx_v)
    pltpu.sync_copy(dst_hbm.at[s], didx_v)
    plsc.subcore_barrier()

    def gather(j, buf, sem):
        @pl.when(c == 0)
        def _():
            pltpu.async_copy(u0_hbm.at[sidx_v.at[j]], buf, sem)

        @pl.when(c == 1)
        def _():
            pltpu.async_copy(u1_hbm.at[sidx_v.at[j]], buf, sem)

    def wait_gather(buf, sem):
        pltpu.make_async_copy(u0_hbm.at[pl.ds(0, CH)], buf, sem).wait()

    gather(0, rows_a, sem_a)

    def pair(jj, carry):
        j0 = 2 * jj
        wait_gather(rows_a, sem_a)
        gather(j0 + 1, rows_b, sem_b)
        pltpu.sync_copy(rows_a, agg_sh.at[didx_v.at[j0]], add=True)
        wait_gather(rows_b, sem_b)

        @pl.when(j0 + 2 < NCHUNK)
        def _():
            gather(j0 + 2, rows_a, sem_a)

        pltpu.sync_copy(rows_b, agg_sh.at[didx_v.at[j0 + 1]], add=True)
        return carry
    lax.fori_loop(0, NCHUNK // 2, pair, 0)

    plsc.subcore_barrier()

    @pl.when(c == 0)
    def _():
        pltpu.sync_copy(agg_sh.at[pl.ds(s * RPT, RPT)],
                        out0_hbm.at[pl.ds(s * RPT, RPT)])

    @pl.when(c == 1)
    def _():
        pltpu.sync_copy(agg_sh.at[pl.ds(s * RPT, RPT)],
                        out1_hbm.at[pl.ds(s * RPT, RPT)])


_HI = lax.Precision.HIGHEST


def _pre_body(deg_ref, x_ref, w_ref, u0_ref, u1_ref, dinv_ref):
    deg = deg_ref[:, 0] + deg_ref[:, 1] + 1.0
    dinv = lax.rsqrt(deg)[:, None]
    hw = jnp.dot(x_ref[...], w_ref[...], precision=_HI,
                 preferred_element_type=jnp.float32)
    u = hw * dinv
    u0_ref[...] = u[:, :DH]
    u1_ref[...] = u[:, DH:]
    dinv_ref[...] = jnp.broadcast_to(dinv, (RB, D))


def _pre_call(deg_pair, x, W):
    return pl.pallas_call(
        _pre_body,
        grid=(NBLK,),
        in_specs=[
            pl.BlockSpec((RB, NC), lambda i: (i, 0)),
            pl.BlockSpec((RB, D), lambda i: (i, 0)),
            pl.BlockSpec((D, D), lambda i: (0, 0)),
        ],
        out_specs=[
            pl.BlockSpec((RB, DH), lambda i: (i, 0)),
            pl.BlockSpec((RB, DH), lambda i: (i, 0)),
            pl.BlockSpec((RB, D), lambda i: (i, 0)),
        ],
        out_shape=[
            jax.ShapeDtypeStruct((N, DH), jnp.float32),
            jax.ShapeDtypeStruct((N, DH), jnp.float32),
            jax.ShapeDtypeStruct((N, D), jnp.float32),
        ],
    )(deg_pair, x, W)


def _post_mix(agg0_ref, agg1_ref, u0_ref, u1_ref, dinv_ref, b_ref, g_ref,
              be_ref):
    dinv = dinv_ref[...]
    msg = jnp.concatenate([agg0_ref[...] + u0_ref[...],
                           agg1_ref[...] + u1_ref[...]], axis=-1)
    t = dinv * msg + b_ref[...]
    mu = jnp.mean(t, axis=-1, keepdims=True)
    var = jnp.mean((t - mu) ** 2, axis=-1, keepdims=True)
    t = (t - mu) / jnp.sqrt(var + 1e-5) * g_ref[...] + be_ref[...]
    return jnp.where(t > 0, t, 0.01 * t)


def _mid_body(agg0_ref, agg1_ref, u0_ref, u1_ref, dinv_ref, b_ref, g_ref,
              be_ref, w_ref, un0_ref, un1_ref):
    h = _post_mix(agg0_ref, agg1_ref, u0_ref, u1_ref, dinv_ref, b_ref,
                  g_ref, be_ref)
    un = dinv_ref[...] * jnp.dot(h, w_ref[...], precision=_HI,
                                 preferred_element_type=jnp.float32)
    un0_ref[...] = un[:, :DH]
    un1_ref[...] = un[:, DH:]


def _mid_call(agg0, agg1, u0, u1, dinv, b, g, be, Wn):
    return pl.pallas_call(
        _mid_body,
        grid=(NBLK,),
        in_specs=[
            pl.BlockSpec((RB, DH), lambda i: (i, 0)),
            pl.BlockSpec((RB, DH), lambda i: (i, 0)),
            pl.BlockSpec((RB, DH), lambda i: (i, 0)),
            pl.BlockSpec((RB, DH), lambda i: (i, 0)),
            pl.BlockSpec((RB, D), lambda i: (i, 0)),
            pl.BlockSpec((1, D), lambda i: (0, 0)),
            pl.BlockSpec((1, D), lambda i: (0, 0)),
            pl.BlockSpec((1, D), lambda i: (0, 0)),
            pl.BlockSpec((D, D), lambda i: (0, 0)),
        ],
        out_specs=[
            pl.BlockSpec((RB, DH), lambda i: (i, 0)),
            pl.BlockSpec((RB, DH), lambda i: (i, 0)),
        ],
        out_shape=[
            jax.ShapeDtypeStruct((N, DH), jnp.float32),
            jax.ShapeDtypeStruct((N, DH), jnp.float32),
        ],
    )(agg0, agg1, u0, u1, dinv, b, g, be, Wn)


def _final_body(agg0_ref, agg1_ref, u0_ref, u1_ref, dinv_ref, b_ref, g_ref,
                be_ref, batch_ref, fw1_ref, fb1_ref, fw2_ref, fb2_ref,
                out_ref, sums, cnts):
    i = pl.program_id(0)
    h = _post_mix(agg0_ref, agg1_ref, u0_ref, u1_ref, dinv_ref, b_ref,
                  g_ref, be_ref)
    bt = batch_ref[0, 0, :]
    mask = (bt[None, :] == lax.broadcasted_iota(jnp.int32, (G, RB), 0))
    mask = mask.astype(jnp.float32)
    psum = jnp.dot(mask, h, precision=_HI, preferred_element_type=jnp.float32)
    pcnt = jnp.broadcast_to(jnp.sum(mask, axis=1)[:, None], (G, D))

    @pl.when(i == 0)
    def _():
        sums[...] = psum
        cnts[...] = pcnt

    @pl.when(i > 0)
    def _():
        sums[...] += psum
        cnts[...] += pcnt

    @pl.when(i == NBLK - 1)
    def _():
        pooled = sums[...] / jnp.maximum(cnts[...], 1.0)
        o = jnp.dot(pooled, fw1_ref[...], precision=_HI,
                    preferred_element_type=jnp.float32) + fb1_ref[...]
        o = jnp.dot(o, fw2_ref[...], precision=_HI,
                    preferred_element_type=jnp.float32) + fb2_ref[...]
        out_ref[...] = o


def _final_call(agg0, agg1, u0, u1, dinv, b, g, be, batch_r, fW1, fb1, fW2,
                fb2):
    return pl.pallas_call(
        _final_body,
        grid=(NBLK,),
        in_specs=[
            pl.BlockSpec((RB, DH), lambda i: (i, 0)),
            pl.BlockSpec((RB, DH), lambda i: (i, 0)),
            pl.BlockSpec((RB, DH), lambda i: (i, 0)),
            pl.BlockSpec((RB, DH), lambda i: (i, 0)),
            pl.BlockSpec((RB, D), lambda i: (i, 0)),
            pl.BlockSpec((1, D), lambda i: (0, 0)),
            pl.BlockSpec((1, D), lambda i: (0, 0)),
            pl.BlockSpec((1, D), lambda i: (0, 0)),
            pl.BlockSpec((1, 1, RB), lambda i: (i, 0, 0)),
            pl.BlockSpec((D, 256), lambda i: (0, 0)),
            pl.BlockSpec((1, 256), lambda i: (0, 0)),
            pl.BlockSpec((256, D), lambda i: (0, 0)),
            pl.BlockSpec((1, D), lambda i: (0, 0)),
        ],
        out_specs=pl.BlockSpec((G, D), lambda i: (0, 0)),
        out_shape=jax.ShapeDtypeStruct((G, D), jnp.float32),
        scratch_shapes=[
            pltpu.VMEM((G, D), jnp.float32),
            pltpu.VMEM((G, D), jnp.float32),
        ],
    )(agg0, agg1, u0, u1, dinv, b, g, be, batch_r, fW1, fb1, fW2, fb2)


def kernel(x, edge_index, batch, W1, b1, W2, b2, W3, b3, g1, be1, g2, be2,
           g3, be3, fW1, fb1, fW2, fb2):
    pad_src = jnp.zeros((EPAD - E,), jnp.int32)
    pad_dst = jnp.full((EPAD - E,), N, jnp.int32)
    src = jnp.concatenate([edge_index[0], pad_src]).reshape(NS, NCHUNK, CH)
    dst = jnp.concatenate([edge_index[1], pad_dst]).reshape(NS, NCHUNK, CH)
    dst_deg = edge_index[1].reshape(NW, NCHUNK_D, CHD)
    batch_r = batch.reshape(NBLK, 1, RB)
    b1r, b2r, b3r = b1.reshape(1, D), b2.reshape(1, D), b3.reshape(1, D)
    g1r, g2r, g3r = g1.reshape(1, D), g2.reshape(1, D), g3.reshape(1, D)
    be1r, be2r, be3r = be1.reshape(1, D), be2.reshape(1, D), be3.reshape(1, D)
    fb1r, fb2r = fb1.reshape(1, 256), fb2.reshape(1, D)

    dm0, dm1 = _deg_kernel(dst_deg)
    deg_pair = jnp.stack([dm0[:N, 0], dm1[:N, 0]], axis=-1)

    u10, u11, dinv = _pre_call(deg_pair, x, W1)
    a10, a11 = _mp_kernel(u10, u11, src, dst)
    u20, u21 = _mid_call(a10, a11, u10, u11, dinv, b1r, g1r, be1r, W2)
    a20, a21 = _mp_kernel(u20, u21, src, dst)
    u30, u31 = _mid_call(a20, a21, u20, u21, dinv, b2r, g2r, be2r, W3)
    a30, a31 = _mp_kernel(u30, u31, src, dst)
    return _final_call(a30, a31, u30, u31, dinv, b3r, g3r, be3r, batch_r,
                       fW1, fb1r, fW2, fb2r)
